# Initial kernel scaffold; baseline (speedup 1.0000x reference)
import jax
import jax.numpy as jnp
from jax.experimental import pallas as pl


def kernel(x, weight):
    raise NotImplementedError("write your pallas kernel here")


# speedup vs baseline: 1.3105x; 1.3105x over previous
import jax
import jax.numpy as jnp
from jax import lax
from jax.experimental import pallas as pl
from jax.experimental.pallas import tpu as pltpu
from jax.experimental.pallas import tpu_sc as plsc

_N = 2048
_L = 16
_S = _N // _L
_NBITS = 8
_NBINS = 1 << _NBITS
_HV = _NBINS // _L
_PASSES = 4
_B = 4
_MINI = -(1 << 31)


def _fspool_body(num_workers, chans_per_worker, core_axis, subcore_axis):
    def body(x_hbm, wflip_hbm, out_hbm, *scratch):
        kbufA = scratch[0:4]
        kbufB = scratch[4:8]
        histA = scratch[8:12]
        histB = scratch[12:16]
        wbuf = scratch[16]
        resbuf = scratch[17]
        bufs = (kbufA, kbufB)
        hists = (histA, histB)

        wid = lax.axis_index(subcore_axis) * 2 + lax.axis_index(core_axis)
        c0 = wid * chans_per_worker

        def chan_body(ci, _):
            c = c0 + ci
            pltpu.sync_copy(wflip_hbm.at[c], wbuf)
            for r in range(_B):
                pltpu.sync_copy(x_hbm.at[r, c], kbufA[r])

            def clr_body(i, _):
                z = jnp.zeros((_L,), jnp.int32)
                for r in range(_B):
                    histA[r][pl.ds(i * _L, _L)] = z
                return 0

            lax.fori_loop(0, _HV, clr_body, 0)

            def prep_body(s, _):
                base = s * _L
                for r in range(_B):
                    v = kbufA[r][pl.ds(base, _L)]
                    u = plsc.bitcast(v, jnp.int32)
                    m = lax.shift_right_arithmetic(u, 31)
                    key = u ^ (m | _MINI)
                    kbufA[r][pl.ds(base, _L)] = plsc.bitcast(key, jnp.float32)
                    d = key & (_NBINS - 1)
                    cnt, last = plsc.scan_count(d)
                    plsc.addupdate_scatter(histA[r], [d], cnt, mask=last)
                return 0

            lax.fori_loop(0, _S, prep_body, 0)

            acc = None
            for p in range(_PASSES):
                sh = p * _NBITS
                src = bufs[p % 2]
                dst = bufs[1 - p % 2]
                hcur = hists[p % 2]
                hnxt = hists[1 - p % 2]
                final = p == _PASSES - 1

                def scan_body(i, carry, hcur=hcur, hnxt=hnxt, final=final):
                    z = jnp.zeros((_L,), jnp.int32)
                    out = []
                    for r in range(_B):
                        v = hcur[r][pl.ds(i * _L, _L)]
                        inc = plsc.cumsum(v)
                        hcur[r][pl.ds(i * _L, _L)] = inc - v + carry[r]
                        out.append(carry[r] + jnp.sum(v))
                        if not final:
                            hnxt[r][pl.ds(i * _L, _L)] = z
                    return tuple(out)

                lax.fori_loop(0, _HV, scan_body, (jnp.int32(-1),) * _B)

                if not final:
                    def perm_body(s, _, src=src, dst=dst, hcur=hcur,
                                  hnxt=hnxt, sh=sh):
                        base = s * _L
                        for r in range(_B):
                            v = src[r][pl.ds(base, _L)]
                            u = plsc.bitcast(v, jnp.int32)
                            d = lax.shift_right_logical(u, sh) & (_NBINS - 1)
                            cnt, last = plsc.scan_count(d)
                            pos = plsc.load_gather(hcur[r], [d]) + cnt
                            plsc.addupdate_scatter(hcur[r], [d], cnt, mask=last)
                            plsc.store_scatter(dst[r], [pos], v)
                            d2 = lax.shift_right_logical(u, sh + _NBITS) \
                                & (_NBINS - 1)
                            cnt2, last2 = plsc.scan_count(d2)
                            plsc.addupdate_scatter(hnxt[r], [d2], cnt2,
                                                   mask=last2)
                        return 0

                    lax.fori_loop(0, _S, perm_body, 0)
                else:
                    def final_body(s, acc, src=src, hcur=hcur, sh=sh):
                        base = s * _L
                        out = []
                        for r in range(_B):
                            v = src[r][pl.ds(base, _L)]
                            u = plsc.bitcast(v, jnp.int32)
                            d = lax.shift_right_logical(u, sh) & (_NBINS - 1)
                            cnt, last = plsc.scan_count(d)
                            pos = plsc.load_gather(hcur[r], [d]) + cnt
                            plsc.addupdate_scatter(hcur[r], [d], cnt,
                                                   mask=last)
                            wv = plsc.load_gather(wbuf, [pos])
                            m2 = lax.shift_right_arithmetic(u, 31)
                            orig = u ^ (~m2 | _MINI)
                            out.append(acc[r]
                                       + plsc.bitcast(orig, jnp.float32) * wv)
                        return tuple(out)

                    acc = lax.fori_loop(
                        0, _S, final_body,
                        tuple(jnp.zeros((_L,), jnp.float32)
                              for _ in range(_B)))

            lane0 = lax.iota(jnp.int32, _L) == 0
            for r in range(_B):
                res = jnp.sum(acc[r])
                idx = jnp.full((_L,), r * chans_per_worker + ci, jnp.int32)
                plsc.store_scatter(resbuf, [idx], jnp.full((_L,), res),
                                   mask=lane0)
            return 0

        lax.fori_loop(0, chans_per_worker, chan_body, 0)

        for r in range(_B):
            pltpu.sync_copy(
                resbuf.at[pl.ds(r * chans_per_worker, chans_per_worker)],
                out_hbm.at[r, pl.ds(c0, chans_per_worker)])

    return body


def kernel(x, weight):
    b, c, n = x.shape
    assert (b, n) == (_B, _N) and weight.shape == (c, n)
    info = plsc.get_sparse_core_info()
    num_workers = info.num_cores * info.num_subcores
    chans_per_worker = c // num_workers
    mesh = plsc.VectorSubcoreMesh(core_axis_name="sc_core",
                                  subcore_axis_name="sc_subcore")
    scratch = (
        [pltpu.VMEM((_N,), jnp.float32) for _ in range(2 * _B)]
        + [pltpu.VMEM((_NBINS,), jnp.int32) for _ in range(2 * _B)]
        + [pltpu.VMEM((_N,), jnp.float32),
           pltpu.VMEM((_B * chans_per_worker,), jnp.float32)]
    )
    k = pl.kernel(
        _fspool_body(num_workers, chans_per_worker, "sc_core", "sc_subcore"),
        out_type=jax.ShapeDtypeStruct((b, c), jnp.float32),
        mesh=mesh,
        scratch_types=scratch,
        compiler_params=pltpu.CompilerParams(needs_layout_passes=False),
    )
    wflip = jnp.flip(weight, axis=1)
    return k(x, wflip)


# speedup vs baseline: 1.3983x; 1.0670x over previous
import jax
import jax.numpy as jnp
from jax import lax
from jax.experimental import pallas as pl
from jax.experimental.pallas import tpu as pltpu
from jax.experimental.pallas import tpu_sc as plsc

_N = 2048
_L = 16
_S = _N // _L
_NBITS = 8
_NBINS = 1 << _NBITS
_HV = _NBINS // _L
_PASSES = 4
_B = 4
_MINI = -(1 << 31)


def _fspool_body(num_workers, chans_per_worker, core_axis, subcore_axis):
    def body(x_hbm, wflip_hbm, out_hbm, *scratch):
        kbufA = scratch[0:4]
        kbufB = scratch[4:8]
        hist = [scratch[8 + 4 * r:8 + 4 * r + 4] for r in range(_B)]
        wbuf = scratch[24]
        resbuf = scratch[25]
        bufs = (kbufA, kbufB)

        wid = lax.axis_index(subcore_axis) * 2 + lax.axis_index(core_axis)
        c0 = wid * chans_per_worker

        def chan_body(ci, _):
            c = c0 + ci
            pltpu.sync_copy(wflip_hbm.at[c], wbuf)
            for r in range(_B):
                pltpu.sync_copy(x_hbm.at[r, c], kbufA[r])

            def clr_body(i, _):
                z = jnp.zeros((_L,), jnp.int32)
                for r in range(_B):
                    for p in range(_PASSES):
                        hist[r][p][pl.ds(i * _L, _L)] = z
                return 0

            lax.fori_loop(0, _HV, clr_body, 0)

            def prep_body(s, _):
                base = s * _L
                ones = jnp.ones((_L,), jnp.int32)
                for r in range(_B):
                    v = kbufA[r][pl.ds(base, _L)]
                    u = plsc.bitcast(v, jnp.int32)
                    m = lax.shift_right_arithmetic(u, 31)
                    key = u ^ (m | _MINI)
                    kbufA[r][pl.ds(base, _L)] = plsc.bitcast(key, jnp.float32)
                    for p in range(_PASSES):
                        d = lax.shift_right_logical(key, p * _NBITS) \
                            & (_NBINS - 1)
                        plsc.addupdate_scatter(hist[r][p], [d], ones)
                return 0

            lax.fori_loop(0, _S, prep_body, 0)

            acc = None
            for p in range(_PASSES):
                sh = p * _NBITS
                src = bufs[p % 2]
                dst = bufs[1 - p % 2]
                hcur = [hist[r][p] for r in range(_B)]
                final = p == _PASSES - 1

                def scan_body(i, carry, hcur=hcur):
                    out = []
                    for r in range(_B):
                        v = hcur[r][pl.ds(i * _L, _L)]
                        inc = plsc.cumsum(v)
                        hcur[r][pl.ds(i * _L, _L)] = inc - v + carry[r]
                        out.append(carry[r] + jnp.sum(v))
                    return tuple(out)

                lax.fori_loop(0, _HV, scan_body, (jnp.int32(-1),) * _B)

                if not final:
                    def perm_body(s, _, src=src, dst=dst, hcur=hcur, sh=sh):
                        base = s * _L
                        for r in range(_B):
                            v = src[r][pl.ds(base, _L)]
                            u = plsc.bitcast(v, jnp.int32)
                            d = lax.shift_right_logical(u, sh) & (_NBINS - 1)
                            cnt, last = plsc.scan_count(d)
                            pos = plsc.load_gather(hcur[r], [d]) + cnt
                            plsc.addupdate_scatter(hcur[r], [d], cnt, mask=last)
                            plsc.store_scatter(dst[r], [pos], v)
                        return 0

                    lax.fori_loop(0, _S, perm_body, 0)
                else:
                    def final_body(s, acc, src=src, hcur=hcur, sh=sh):
                        base = s * _L
                        out = []
                        for r in range(_B):
                            v = src[r][pl.ds(base, _L)]
                            u = plsc.bitcast(v, jnp.int32)
                            d = lax.shift_right_logical(u, sh) & (_NBINS - 1)
                            cnt, last = plsc.scan_count(d)
                            pos = plsc.load_gather(hcur[r], [d]) + cnt
                            plsc.addupdate_scatter(hcur[r], [d], cnt,
                                                   mask=last)
                            wv = plsc.load_gather(wbuf, [pos])
                            m2 = lax.shift_right_arithmetic(u, 31)
                            orig = u ^ (~m2 | _MINI)
                            out.append(acc[r]
                                       + plsc.bitcast(orig, jnp.float32) * wv)
                        return tuple(out)

                    acc = lax.fori_loop(
                        0, _S, final_body,
                        tuple(jnp.zeros((_L,), jnp.float32)
                              for _ in range(_B)))

            lane0 = lax.iota(jnp.int32, _L) == 0
            for r in range(_B):
                res = jnp.sum(acc[r])
                idx = jnp.full((_L,), r * chans_per_worker + ci, jnp.int32)
                plsc.store_scatter(resbuf, [idx], jnp.full((_L,), res),
                                   mask=lane0)
            return 0

        lax.fori_loop(0, chans_per_worker, chan_body, 0)

        for r in range(_B):
            pltpu.sync_copy(
                resbuf.at[pl.ds(r * chans_per_worker, chans_per_worker)],
                out_hbm.at[r, pl.ds(c0, chans_per_worker)])

    return body


def kernel(x, weight):
    b, c, n = x.shape
    assert (b, n) == (_B, _N) and weight.shape == (c, n)
    info = plsc.get_sparse_core_info()
    num_workers = info.num_cores * info.num_subcores
    chans_per_worker = c // num_workers
    mesh = plsc.VectorSubcoreMesh(core_axis_name="sc_core",
                                  subcore_axis_name="sc_subcore")
    scratch = (
        [pltpu.VMEM((_N,), jnp.float32) for _ in range(2 * _B)]
        + [pltpu.VMEM((_NBINS,), jnp.int32) for _ in range(_PASSES * _B)]
        + [pltpu.VMEM((_N,), jnp.float32),
           pltpu.VMEM((_B * chans_per_worker,), jnp.float32)]
    )
    k = pl.kernel(
        _fspool_body(num_workers, chans_per_worker, "sc_core", "sc_subcore"),
        out_type=jax.ShapeDtypeStruct((b, c), jnp.float32),
        mesh=mesh,
        scratch_types=scratch,
        compiler_params=pltpu.CompilerParams(needs_layout_passes=False),
    )
    wflip = jnp.flip(weight, axis=1)
    return k(x, wflip)
